# baseline (speedup 1.0000x reference)
import functools

import jax
import jax.numpy as jnp
from jax import lax
from jax.experimental import pallas as pl
from jax.experimental.pallas import tpu as pltpu
from jax.experimental.pallas import tpu_sc as plsc

NC = 2
NS = 16
NW = NC * NS
L = 16

B, H, V, D = 16384, 200, 25, 32
DT = D // 8
BB = B // 128
SLABS = H * DT
SLABS_PER_W = SLABS // NW
WT_PER_CHUNK = 16
CHUNKS = BB // WT_PER_CHUNK
B_PER_CHUNK = WT_PER_CHUNK * 128


def kernel(positions, table):
    pos_t = jnp.maximum(positions, jnp.int32(0)).T

    mesh = plsc.VectorSubcoreMesh(
        core_axis_name="c", subcore_axis_name="s", num_cores=NC, num_subcores=NS
    )

    @functools.partial(
        pl.kernel,
        out_type=jax.ShapeDtypeStruct((H, DT, BB, 8, 128), jnp.float32),
        mesh=mesh,
        scratch_types=[
            pltpu.VMEM((V, D), jnp.float32),
            pltpu.VMEM((B,), jnp.int32),
            pltpu.VMEM((2, WT_PER_CHUNK, 8, 128), jnp.float32),
            pltpu.SemaphoreType.DMA((2,)),
        ],
        compiler_params=pltpu.CompilerParams(use_tc_tiling_on_sc=False,
                                             needs_layout_passes=False),
    )
    def gather_kernel(pos_hbm, table_hbm, out_hbm, table_v, idxrow_v,
                      stage_v, wsem):
        cid = lax.axis_index("c")
        sid = lax.axis_index("s")
        wid = sid * NC + cid

        pltpu.sync_copy(table_hbm, table_v)

        def wb_descriptor(h, dt, c, buf):
            return pltpu.make_async_copy(
                stage_v.at[buf],
                out_hbm.at[h, dt, pl.ds(c * WT_PER_CHUNK, WT_PER_CHUNK)],
                wsem.at[buf])

        def do_slab(j, carry):
            s = wid * SLABS_PER_W + j
            h = s // DT
            dt = s % DT

            @pl.when(jnp.logical_or(j == 0, dt == 0))
            def _():
                pltpu.sync_copy(pos_hbm.at[h], idxrow_v)

            dcol = [jnp.full((L,), dt * 8 + d, jnp.int32) for d in range(8)]

            def do_chunk(c, carry2):
                buf = c % 2

                @pl.when(j * CHUNKS + c >= 2)
                def _():
                    wb_descriptor(h, dt, c, buf).wait()

                def do_b16(i, carry3):
                    base = c * B_PER_CHUNK + i * 2 * L
                    for u in range(2):
                        g = i * 2 + u
                        idxv = idxrow_v[pl.ds(base + u * L, L)]
                        w = g // 8
                        lgrp = g % 8
                        for d in range(8):
                            vals = plsc.load_gather(table_v, [idxv, dcol[d]])
                            stage_v[buf, w, d, pl.ds(lgrp * L, L)] = vals
                    return carry3

                lax.fori_loop(0, B_PER_CHUNK // (2 * L), do_b16, 0,
                              unroll=False)

                pltpu.async_copy(
                    stage_v.at[buf],
                    out_hbm.at[h, dt, pl.ds(c * WT_PER_CHUNK, WT_PER_CHUNK)],
                    wsem.at[buf])
                return carry2

            lax.fori_loop(0, CHUNKS, do_chunk, carry, unroll=False)
            return carry

        lax.fori_loop(0, SLABS_PER_W, do_slab, jnp.int32(0), unroll=False)

        last = wid * SLABS_PER_W + SLABS_PER_W - 1
        for c in (CHUNKS - 2, CHUNKS - 1):
            wb_descriptor(last // DT, last % DT, c, c % 2).wait()

    p = gather_kernel(pos_t, table)
    return p.transpose(2, 4, 0, 1, 3).reshape(B, H, D)


# speedup vs baseline: 1.0024x; 1.0024x over previous
import functools

import jax
import jax.numpy as jnp
from jax import lax
from jax.experimental import pallas as pl
from jax.experimental.pallas import tpu as pltpu
from jax.experimental.pallas import tpu_sc as plsc

NC = 2
NS = 16
NW = NC * NS
L = 16

B, H, V, D = 16384, 200, 25, 32
DT = D // 8
BB = B // 128
SLABS = H * DT
SLAB_ELEMS = BB * 8 * 128
SLABS_PER_W = SLABS // NW
WT_PER_CHUNK = 16
CHUNK_ELEMS = WT_PER_CHUNK * 1024
CHUNKS = BB // WT_PER_CHUNK
B_PER_CHUNK = WT_PER_CHUNK * 128


def kernel(positions, table):
    pos_t = jnp.maximum(positions, jnp.int32(0)).T

    mesh = plsc.VectorSubcoreMesh(
        core_axis_name="c", subcore_axis_name="s", num_cores=NC, num_subcores=NS
    )

    @functools.partial(
        pl.kernel,
        out_type=jax.ShapeDtypeStruct((H, DT, SLAB_ELEMS), jnp.float32),
        mesh=mesh,
        scratch_types=[
            pltpu.VMEM((V, D), jnp.float32),
            pltpu.VMEM((B,), jnp.int32),
            pltpu.VMEM((2, CHUNK_ELEMS), jnp.float32),
            pltpu.SemaphoreType.DMA((2,)),
        ],
        compiler_params=pltpu.CompilerParams(use_tc_tiling_on_sc=False,
                                             needs_layout_passes=False),
    )
    def gather_kernel(pos_hbm, table_hbm, out_hbm, table_v, idxrow_v,
                      stage_v, wsem):
        cid = lax.axis_index("c")
        sid = lax.axis_index("s")
        wid = sid * NC + cid

        pltpu.sync_copy(table_hbm, table_v)

        def wb_descriptor(h, dt, c, buf):
            return pltpu.make_async_copy(
                stage_v.at[buf],
                out_hbm.at[h, dt, pl.ds(c * CHUNK_ELEMS, CHUNK_ELEMS)],
                wsem.at[buf])

        def do_slab(j, carry):
            s = wid * SLABS_PER_W + j
            h = s // DT
            dt = s % DT

            @pl.when(jnp.logical_or(j == 0, dt == 0))
            def _():
                pltpu.sync_copy(pos_hbm.at[h], idxrow_v)

            dcol = [jnp.full((L,), dt * 8 + d, jnp.int32) for d in range(8)]

            def do_chunk(c, carry2):
                buf = c % 2

                @pl.when(j * CHUNKS + c >= 2)
                def _():
                    wb_descriptor(h, dt, c, buf).wait()

                cbase = c * B_PER_CHUNK

                def do_wt(w, carry3):
                    wbase = w * 1024
                    ibase = cbase + w * 128
                    for lgrp in range(8):
                        idxv = idxrow_v[pl.ds(ibase + lgrp * L, L)]
                        for d in range(8):
                            vals = plsc.load_gather(table_v, [idxv, dcol[d]])
                            stage_v[buf,
                                    pl.ds(wbase + d * 128 + lgrp * L, L)] = vals
                    return carry3

                lax.fori_loop(0, WT_PER_CHUNK, do_wt, 0, unroll=False)

                pltpu.async_copy(
                    stage_v.at[buf],
                    out_hbm.at[h, dt, pl.ds(c * CHUNK_ELEMS, CHUNK_ELEMS)],
                    wsem.at[buf])
                return carry2

            lax.fori_loop(0, CHUNKS, do_chunk, carry, unroll=False)
            return carry

        lax.fori_loop(0, SLABS_PER_W, do_slab, jnp.int32(0), unroll=False)

        last = wid * SLABS_PER_W + SLABS_PER_W - 1
        for c in (CHUNKS - 2, CHUNKS - 1):
            wb_descriptor(last // DT, last % DT, c, c % 2).wait()

    p = gather_kernel(pos_t, table)
    return (p.reshape(H, DT, BB, 8, 128)
            .transpose(2, 4, 0, 1, 3).reshape(B, H, D))


# speedup vs baseline: 13.9990x; 13.9652x over previous
import functools

import jax
import jax.numpy as jnp
from jax import lax
from jax.experimental import pallas as pl
from jax.experimental.pallas import tpu as pltpu
from jax.experimental.pallas import tpu_sc as plsc

NC = 2
NS = 16
NW = NC * NS
L = 16

B, H, V, D = 16384, 200, 25, 32
DT = D // 8
BB = B // 128
SLABS = H * DT
SLAB_ELEMS = BB * 8 * 128
SLABS_PER_W = SLABS // NW
WT_PER_CHUNK = 16
CHUNK_ELEMS = WT_PER_CHUNK * 1024
CHUNKS = BB // WT_PER_CHUNK
B_PER_CHUNK = WT_PER_CHUNK * 128


def kernel(positions, table):
    pos_t = jnp.maximum(positions, jnp.int32(0)).T

    mesh = plsc.VectorSubcoreMesh(
        core_axis_name="c", subcore_axis_name="s", num_cores=NC, num_subcores=NS
    )

    @functools.partial(
        pl.kernel,
        out_type=jax.ShapeDtypeStruct((H, DT, SLAB_ELEMS), jnp.float32),
        mesh=mesh,
        scratch_types=[
            pltpu.VMEM((V, D), jnp.float32),
            pltpu.VMEM((B,), jnp.int32),
            pltpu.VMEM((2, CHUNK_ELEMS), jnp.float32),
            pltpu.SemaphoreType.DMA((2,)),
        ],
        compiler_params=pltpu.CompilerParams(use_tc_tiling_on_sc=False,
                                             needs_layout_passes=False),
    )
    def gather_kernel(pos_hbm, table_hbm, out_hbm, table_v, idxrow_v,
                      stage_v, wsem):
        cid = lax.axis_index("c")
        sid = lax.axis_index("s")
        wid = sid * NC + cid

        pltpu.sync_copy(table_hbm, table_v)

        def wb_descriptor(h, dt, c, buf):
            return pltpu.make_async_copy(
                stage_v.at[buf],
                out_hbm.at[h, dt, pl.ds(c * CHUNK_ELEMS, CHUNK_ELEMS)],
                wsem.at[buf])

        def do_slab(j, carry):
            s = wid * SLABS_PER_W + j
            h = s // DT
            dt = s % DT

            @pl.when(jnp.logical_or(j == 0, dt == 0))
            def _():
                pltpu.sync_copy(pos_hbm.at[h], idxrow_v)

            dcol = [jnp.full((L,), dt * 8 + d, jnp.int32) for d in range(8)]

            def do_chunk(c, carry2):
                buf = c % 2

                @pl.when(j * CHUNKS + c >= 2)
                def _():
                    wb_descriptor(h, dt, c, buf).wait()

                cbase = c * B_PER_CHUNK

                @functools.partial(plsc.parallel_loop, 0, WT_PER_CHUNK,
                                   unroll=2)
                def _(w):
                    wbase = w * 1024
                    ibase = cbase + w * 128
                    for lgrp in range(8):
                        idxv = idxrow_v[pl.ds(ibase + lgrp * L, L)]
                        for d in range(8):
                            vals = plsc.load_gather(table_v, [idxv, dcol[d]])
                            stage_v[buf,
                                    pl.ds(wbase + d * 128 + lgrp * L, L)] = vals

                pltpu.async_copy(
                    stage_v.at[buf],
                    out_hbm.at[h, dt, pl.ds(c * CHUNK_ELEMS, CHUNK_ELEMS)],
                    wsem.at[buf])
                return carry2

            lax.fori_loop(0, CHUNKS, do_chunk, carry, unroll=False)
            return carry

        lax.fori_loop(0, SLABS_PER_W, do_slab, jnp.int32(0), unroll=False)

        last = wid * SLABS_PER_W + SLABS_PER_W - 1
        for c in (CHUNKS - 2, CHUNKS - 1):
            wb_descriptor(last // DT, last % DT, c, c % 2).wait()

    p = gather_kernel(pos_t, table)
    return (p.reshape(H, DT, BB, 8, 128)
            .transpose(2, 4, 0, 1, 3).reshape(B, H, D))
